# baseline (speedup 1.0000x reference)
import jax
import jax.numpy as jnp
from jax.experimental import pallas as pl
from jax.experimental.pallas import tpu as pltpu


def _loss_kernel(idx_ref, len_ref, table_ref, out_ref, va, vb, sem_a, sem_b):
    ia = idx_ref[0]
    ib = idx_ref[1]
    cp_a = pltpu.make_async_copy(table_ref.at[pl.ds(ia, 1), :], va, sem_a)
    cp_b = pltpu.make_async_copy(table_ref.at[pl.ds(ib, 1), :], vb, sem_b)
    cp_a.start()
    cp_b.start()
    cp_a.wait()
    cp_b.wait()
    d = jnp.abs(va[...] - vb[...])
    col = jax.lax.broadcasted_iota(jnp.int32, (1, 64), 1)
    d = jnp.where(col < 50, d, jnp.float32(jnp.inf))
    min_d = jnp.min(d)
    ls = len_ref[0]
    l1 = jnp.abs(min_d - ls)
    out_ref[0] = jnp.where(min_d < ls, jnp.float32(100.0) * l1, l1)


def kernel(index_vec, neighbor_index_vec, len_sum, table):
    idx = jnp.stack([index_vec[0], neighbor_index_vec[0]]).astype(jnp.int32)
    len_arr = jnp.reshape(len_sum, (1,))
    out = pl.pallas_call(
        _loss_kernel,
        in_specs=[
            pl.BlockSpec(memory_space=pltpu.SMEM),
            pl.BlockSpec(memory_space=pltpu.SMEM),
            pl.BlockSpec(memory_space=pl.ANY),
        ],
        out_specs=pl.BlockSpec(memory_space=pltpu.SMEM),
        out_shape=jax.ShapeDtypeStruct((1,), jnp.float32),
        scratch_shapes=[
            pltpu.VMEM((1, 64), jnp.float32),
            pltpu.VMEM((1, 64), jnp.float32),
            pltpu.SemaphoreType.DMA,
            pltpu.SemaphoreType.DMA,
        ],
    )(idx, len_arr, table)
    return out[0]


# speedup vs baseline: 112.5167x; 112.5167x over previous
import jax
import jax.numpy as jnp
from jax.experimental import pallas as pl
from jax.experimental.pallas import tpu as pltpu

_E = 1000000


def _col_select(block, col, col_iota):
    return jnp.min(jnp.where(col_iota == col, block, jnp.float32(jnp.inf)),
                   axis=1, keepdims=True)


def _loss_kernel(idx_ref, nidx_ref, len_ref, tt_ref, out_ref,
                 si, sn, va, vb, sem_i, sem_n, sem_a, sem_b):
    cp_i = pltpu.make_async_copy(idx_ref.at[pl.ds(0, 128)], si, sem_i)
    cp_n = pltpu.make_async_copy(nidx_ref.at[pl.ds(0, 128)], sn, sem_n)
    cp_i.start()
    cp_n.start()
    cp_i.wait()
    cp_n.wait()
    ia = si[0]
    ib = sn[0]
    sa = pl.multiple_of((ia // 128) * 128, 128)
    sb = pl.multiple_of((ib // 128) * 128, 128)
    cp_a = pltpu.make_async_copy(tt_ref.at[:, pl.ds(sa, 128)], va, sem_a)
    cp_b = pltpu.make_async_copy(tt_ref.at[:, pl.ds(sb, 128)], vb, sem_b)
    cp_a.start()
    cp_b.start()
    cp_a.wait()
    cp_b.wait()
    col_iota = jax.lax.broadcasted_iota(jnp.int32, (64, 128), 1)
    a = _col_select(va[...], ia - sa, col_iota)
    b = _col_select(vb[...], ib - sb, col_iota)
    d = jnp.abs(a - b)
    row_iota = jax.lax.broadcasted_iota(jnp.int32, (64, 1), 0)
    d = jnp.where(row_iota < 50, d, jnp.float32(jnp.inf))
    min_d = jnp.min(d)
    ls = len_ref[0]
    l1 = jnp.abs(min_d - ls)
    out_ref[0] = jnp.where(min_d < ls, jnp.float32(100.0) * l1, l1)


def kernel(index_vec, neighbor_index_vec, len_sum, table):
    tt = table.T
    len_arr = jnp.reshape(len_sum, (1,))
    out = pl.pallas_call(
        _loss_kernel,
        in_specs=[
            pl.BlockSpec(memory_space=pl.ANY),
            pl.BlockSpec(memory_space=pl.ANY),
            pl.BlockSpec(memory_space=pltpu.SMEM),
            pl.BlockSpec(memory_space=pl.ANY),
        ],
        out_specs=pl.BlockSpec(memory_space=pltpu.SMEM),
        out_shape=jax.ShapeDtypeStruct((1,), jnp.float32),
        scratch_shapes=[
            pltpu.SMEM((128,), jnp.int32),
            pltpu.SMEM((128,), jnp.int32),
            pltpu.VMEM((64, 128), jnp.float32),
            pltpu.VMEM((64, 128), jnp.float32),
            pltpu.SemaphoreType.DMA,
            pltpu.SemaphoreType.DMA,
            pltpu.SemaphoreType.DMA,
            pltpu.SemaphoreType.DMA,
        ],
    )(index_vec.astype(jnp.int32), neighbor_index_vec.astype(jnp.int32),
      len_arr, tt)
    return out[0]
